# baseline (speedup 1.0000x reference)
import functools

import jax
import jax.numpy as jnp
from jax import lax
from jax.experimental import pallas as pl
from jax.experimental.pallas import tpu as pltpu
from jax.experimental.pallas import tpu_sc as plsc

B = 16384
D = 64
NC = 2
NS = 16
NW = NC * NS
BPW = B // NW
CH = 128
NCH = BPW // CH
L = 16
PAD = L + 1

_mesh = plsc.VectorSubcoreMesh(core_axis_name="c", subcore_axis_name="s")


@functools.partial(
    pl.kernel,
    out_type=jax.ShapeDtypeStruct((B,), jnp.float32),
    mesh=_mesh,
    compiler_params=pltpu.CompilerParams(
        needs_layout_passes=False, use_tc_tiling_on_sc=False
    ),
    scratch_types=[
        pltpu.VMEM((2 * BPW,), jnp.int32),
        pltpu.VMEM((NCH, CH), jnp.int32),
        pltpu.VMEM((NCH, CH), jnp.int32),
        pltpu.VMEM((BPW, D), jnp.float32),
        pltpu.VMEM((BPW, D), jnp.float32),
        pltpu.VMEM((L * PAD,), jnp.float32),
        pltpu.VMEM((BPW,), jnp.float32),
        pltpu.SemaphoreType.DMA,
        pltpu.SemaphoreType.DMA,
    ],
)
def _sc_dual_gather_dot(pairs_hbm, user_hbm, item_hbm, out_hbm,
                        pairs_v, uidx_v, iidx_v, urows, irows, tmat, outv,
                        usem, isem):
    wid = lax.axis_index("s") * NC + lax.axis_index("c")
    base = wid * BPW

    pltpu.sync_copy(pairs_hbm.at[wid], pairs_v)
    iota = lax.iota(jnp.int32, L)
    iota2 = iota * 2
    for j in range(NCH):
        for k in range(CH // L):
            off = (j * CH + k * L) * 2
            uidx_v[j, pl.ds(k * L, L)] = plsc.load_gather(
                pairs_v, [iota2 + off])
            iidx_v[j, pl.ds(k * L, L)] = plsc.load_gather(
                pairs_v, [iota2 + (off + 1)])

    copies = [
        pltpu.async_copy(user_hbm.at[uidx_v.at[j]],
                         urows.at[pl.ds(j * CH, CH)], usem)
        for j in range(NCH)
    ] + [
        pltpu.async_copy(item_hbm.at[iidx_v.at[j]],
                         irows.at[pl.ds(j * CH, CH)], isem)
        for j in range(NCH)
    ]
    for c in copies:
        c.wait()

    gather_idx = [iota * PAD + l for l in range(L)]

    def block_body(blk, _):
        rbase = blk * L
        for j in range(L):
            b = rbase + j
            s = urows[b, pl.ds(0, L)] * irows[b, pl.ds(0, L)]
            for d0 in range(L, D, L):
                s = s + urows[b, pl.ds(d0, L)] * irows[b, pl.ds(d0, L)]
            tmat[pl.ds(j * PAD, L)] = s
        acc = plsc.load_gather(tmat, [gather_idx[0]])
        for l in range(1, L):
            acc = acc + plsc.load_gather(tmat, [gather_idx[l]])
        outv[pl.ds(rbase, L)] = acc
        return 0

    lax.fori_loop(0, BPW // L, block_body, 0)

    pltpu.sync_copy(outv, out_hbm.at[pl.ds(base, BPW)])


def kernel(inputs, user_table, item_table):
    pairs = inputs.reshape(NW, 2 * BPW)
    return _sc_dual_gather_dot(pairs, user_table, item_table)


# speedup vs baseline: 1.0477x; 1.0477x over previous
import functools

import jax
import jax.numpy as jnp
from jax import lax
from jax.experimental import pallas as pl
from jax.experimental.pallas import tpu as pltpu
from jax.experimental.pallas import tpu_sc as plsc

B = 16384
D = 64
DP = 128
NC = 2
NS = 16
NW = NC * NS
BPW = B // NW
CH = 128
NCH = BPW // CH
L = 16
PAD = L + 1

_mesh = plsc.VectorSubcoreMesh(core_axis_name="c", subcore_axis_name="s")


@functools.partial(
    pl.kernel,
    out_type=jax.ShapeDtypeStruct((B,), jnp.float32),
    mesh=_mesh,
    compiler_params=pltpu.CompilerParams(
        needs_layout_passes=False, use_tc_tiling_on_sc=False
    ),
    scratch_types=[
        pltpu.VMEM((2 * BPW,), jnp.int32),
        pltpu.VMEM((NCH, CH), jnp.int32),
        pltpu.VMEM((NCH, CH), jnp.int32),
        pltpu.VMEM((2, CH, DP), jnp.float32),
        pltpu.VMEM((2, CH, DP), jnp.float32),
        pltpu.VMEM((L * PAD,), jnp.float32),
        pltpu.VMEM((BPW,), jnp.float32),
        pltpu.SemaphoreType.DMA,
        pltpu.SemaphoreType.DMA,
        pltpu.SemaphoreType.DMA,
        pltpu.SemaphoreType.DMA,
    ],
)
def _sc_dual_gather_dot(pairs_hbm, user_hbm, item_hbm, out_hbm,
                        pairs_v, uidx_v, iidx_v, urows, irows, tmat, outv,
                        usem0, usem1, isem0, isem1):
    wid = lax.axis_index("s") * NC + lax.axis_index("c")
    base = wid * BPW

    pltpu.sync_copy(pairs_hbm.at[wid], pairs_v)
    iota = lax.iota(jnp.int32, L)
    iota2 = iota * 2
    for j in range(NCH):
        for k in range(CH // L):
            off = (j * CH + k * L) * 2
            uidx_v[j, pl.ds(k * L, L)] = plsc.load_gather(
                pairs_v, [iota2 + off])
            iidx_v[j, pl.ds(k * L, L)] = plsc.load_gather(
                pairs_v, [iota2 + (off + 1)])

    usems = [usem0, usem1]
    isems = [isem0, isem1]

    def fire(q):
        buf = q % 2
        cu = pltpu.async_copy(user_hbm.at[uidx_v.at[q]],
                              urows.at[buf], usems[buf])
        ci = pltpu.async_copy(item_hbm.at[iidx_v.at[q]],
                              irows.at[buf], isems[buf])
        return cu, ci

    gather_idx = [iota * PAD + l for l in range(L)]

    def compute_chunk(q):
        buf = q % 2

        def block_body(blk, _):
            rbase = blk * L
            for j in range(L):
                b = rbase + j
                s = (urows[buf, b, pl.ds(0, L)]
                     * irows[buf, b, pl.ds(0, L)])
                for d0 in range(L, D, L):
                    s = s + (urows[buf, b, pl.ds(d0, L)]
                             * irows[buf, b, pl.ds(d0, L)])
                tmat[pl.ds(j * PAD, L)] = s
            acc = plsc.load_gather(tmat, [gather_idx[0]])
            for l in range(1, L):
                acc = acc + plsc.load_gather(tmat, [gather_idx[l]])
            outv[pl.ds(q * CH + rbase, L)] = acc
            return 0

        lax.fori_loop(0, CH // L, block_body, 0)

    inflight = fire(0)
    for q in range(NCH):
        nxt = fire(q + 1) if q + 1 < NCH else None
        inflight[0].wait()
        inflight[1].wait()
        compute_chunk(q)
        inflight = nxt

    pltpu.sync_copy(outv, out_hbm.at[pl.ds(base, BPW)])


def kernel(inputs, user_table, item_table):
    pairs = inputs.reshape(NW, 2 * BPW)
    up = jnp.pad(user_table, ((0, 0), (0, DP - D)))
    ip = jnp.pad(item_table, ((0, 0), (0, DP - D)))
    return _sc_dual_gather_dot(pairs, up, ip)


# speedup vs baseline: 1.0482x; 1.0005x over previous
import functools

import jax
import jax.numpy as jnp
from jax import lax
from jax.experimental import pallas as pl
from jax.experimental.pallas import tpu as pltpu
from jax.experimental.pallas import tpu_sc as plsc

B = 16384
D = 64
DP = 128
NC = 2
NS = 16
NW = NC * NS
BPW = B // NW
CH = 128
NCH = BPW // CH
L = 16
PAD = L + 1

_mesh = plsc.VectorSubcoreMesh(core_axis_name="c", subcore_axis_name="s")


@functools.partial(
    pl.kernel,
    out_type=jax.ShapeDtypeStruct((B,), jnp.float32),
    mesh=_mesh,
    compiler_params=pltpu.CompilerParams(
        needs_layout_passes=False, use_tc_tiling_on_sc=True
    ),
    scratch_types=[
        pltpu.VMEM((2 * BPW,), jnp.int32),
        pltpu.VMEM((NCH, CH), jnp.int32),
        pltpu.VMEM((NCH, CH), jnp.int32),
        pltpu.VMEM((2, CH, DP), jnp.float32),
        pltpu.VMEM((2, CH, DP), jnp.float32),
        pltpu.VMEM((L * PAD,), jnp.float32),
        pltpu.VMEM((BPW,), jnp.float32),
        pltpu.SemaphoreType.DMA,
        pltpu.SemaphoreType.DMA,
        pltpu.SemaphoreType.DMA,
        pltpu.SemaphoreType.DMA,
    ],
)
def _sc_dual_gather_dot(pairs_hbm, user_hbm, item_hbm, out_hbm,
                        pairs_v, uidx_v, iidx_v, urows, irows, tmat, outv,
                        usem0, usem1, isem0, isem1):
    wid = lax.axis_index("s") * NC + lax.axis_index("c")
    base = wid * BPW

    pltpu.sync_copy(pairs_hbm.at[wid], pairs_v)
    iota = lax.iota(jnp.int32, L)
    iota2 = iota * 2
    for j in range(NCH):
        for k in range(CH // L):
            off = (j * CH + k * L) * 2
            uidx_v[j, pl.ds(k * L, L)] = plsc.load_gather(
                pairs_v, [iota2 + off])
            iidx_v[j, pl.ds(k * L, L)] = plsc.load_gather(
                pairs_v, [iota2 + (off + 1)])

    usems = [usem0, usem1]
    isems = [isem0, isem1]

    def fire(q):
        buf = q % 2
        cu = pltpu.async_copy(user_hbm.at[uidx_v.at[q]],
                              urows.at[buf], usems[buf])
        ci = pltpu.async_copy(item_hbm.at[iidx_v.at[q]],
                              irows.at[buf], isems[buf])
        return cu, ci

    gather_idx = [iota * PAD + l for l in range(L)]

    def compute_chunk(q):
        buf = q % 2

        def block_body(blk, _):
            rbase = blk * L
            for j in range(L):
                b = rbase + j
                s = (urows[buf, b, pl.ds(0, L)]
                     * irows[buf, b, pl.ds(0, L)])
                for d0 in range(L, D, L):
                    s = s + (urows[buf, b, pl.ds(d0, L)]
                             * irows[buf, b, pl.ds(d0, L)])
                tmat[pl.ds(j * PAD, L)] = s
            acc = plsc.load_gather(tmat, [gather_idx[0]])
            for l in range(1, L):
                acc = acc + plsc.load_gather(tmat, [gather_idx[l]])
            outv[pl.ds(q * CH + rbase, L)] = acc
            return 0

        lax.fori_loop(0, CH // L, block_body, 0)

    inflight = fire(0)
    for q in range(NCH):
        nxt = fire(q + 1) if q + 1 < NCH else None
        inflight[0].wait()
        inflight[1].wait()
        compute_chunk(q)
        inflight = nxt

    pltpu.sync_copy(outv, out_hbm.at[pl.ds(base, BPW)])


def kernel(inputs, user_table, item_table):
    pairs = inputs.reshape(NW, 2 * BPW)
    up = jnp.pad(user_table, ((0, 0), (0, DP - D)))
    ip = jnp.pad(item_table, ((0, 0), (0, DP - D)))
    return _sc_dual_gather_dot(pairs, up, ip)
